# Initial kernel scaffold; baseline (speedup 1.0000x reference)
import jax
import jax.numpy as jnp
from jax.experimental import pallas as pl


def kernel(graph, coverpoint, coverpoint_mask, batch_xs, batch_as, W_in, b_in, W1, b1, W2, b2):
    raise NotImplementedError("write your pallas kernel here")


# speedup vs baseline: 2.7517x; 2.7517x over previous
import jax
import jax.numpy as jnp
from jax.experimental import pallas as pl
from jax.experimental.pallas import tpu as pltpu


def _cdfg_kernel(idx_ref, xs_ref, a_ref, m_ref, win_ref, bin_ref,
                 w1_ref, b1_ref, w2_ref, b2_ref, out_ref):
    b = pl.program_id(0)
    xs = xs_ref[0]
    a = a_ref[0]
    m = m_ref[b][None, :]
    x0 = jnp.maximum(
        jnp.dot(xs, win_ref[...], preferred_element_type=jnp.float32)
        + bin_ref[...], 0.0)
    y1 = jnp.dot(x0, w1_ref[...], preferred_element_type=jnp.float32)
    x1 = jnp.maximum(
        jnp.dot(a, y1, preferred_element_type=jnp.float32) + b1_ref[...], 0.0)
    y2 = jnp.dot(x1, w2_ref[...], preferred_element_type=jnp.float32)
    x2 = jnp.tanh(
        jnp.dot(a, y2, preferred_element_type=jnp.float32) + b2_ref[...])
    x = x2 + x0
    num = jnp.dot(m, x, preferred_element_type=jnp.float32)
    den = jnp.sum(m)
    out_ref[b, :] = (num / den)[0]


def kernel(graph, coverpoint, coverpoint_mask, batch_xs, batch_as,
           W_in, b_in, W1, b1, W2, b2):
    B = graph.shape[0]
    _, N, F = batch_xs.shape
    H = W1.shape[1]
    idx = graph[:, 0].astype(jnp.int32)
    m = coverpoint_mask.astype(jnp.float32)

    grid_spec = pltpu.PrefetchScalarGridSpec(
        num_scalar_prefetch=1,
        grid=(B,),
        in_specs=[
            pl.BlockSpec((1, N, F), lambda b, i: (i[b], 0, 0)),
            pl.BlockSpec((1, N, N), lambda b, i: (i[b], 0, 0)),
            pl.BlockSpec((B, N), lambda b, i: (0, 0)),
            pl.BlockSpec((F, H), lambda b, i: (0, 0)),
            pl.BlockSpec((1, H), lambda b, i: (0, 0)),
            pl.BlockSpec((H, H), lambda b, i: (0, 0)),
            pl.BlockSpec((1, H), lambda b, i: (0, 0)),
            pl.BlockSpec((H, H), lambda b, i: (0, 0)),
            pl.BlockSpec((1, H), lambda b, i: (0, 0)),
        ],
        out_specs=pl.BlockSpec((B, H), lambda b, i: (0, 0)),
    )
    return pl.pallas_call(
        _cdfg_kernel,
        grid_spec=grid_spec,
        out_shape=jax.ShapeDtypeStruct((B, H), jnp.float32),
        compiler_params=pltpu.CompilerParams(
            vmem_limit_bytes=100 * 1024 * 1024),
    )(idx, batch_xs, batch_as, m,
      W_in, b_in.reshape(1, -1), W1, b1.reshape(1, -1), W2, b2.reshape(1, -1))
